# Initial kernel scaffold; baseline (speedup 1.0000x reference)
import jax
import jax.numpy as jnp
from jax.experimental import pallas as pl


def kernel(x, edge_index, edge_weight, batch, W_in, W_gat, att_src, att_dst, W_edge, att_edge, b_gat, kan1, kan2, W_out):
    raise NotImplementedError("write your pallas kernel here")


# speedup vs baseline: 56.7111x; 56.7111x over previous
import functools

import jax
import jax.numpy as jnp
from jax import lax
from jax.experimental import pallas as pl
from jax.experimental.pallas import tpu as pltpu
from jax.experimental.pallas import tpu_sc as plsc

H = 8
C = 16
G = 16
D = 128
DE = 144
NG = 64

N_PAD = 10112
ROWS = 128
N_TILES = N_PAD // ROWS
ROWS_SC = N_PAD // 16

NW = 32
K = 128
CH = 81
T = K * CH
E_PAD = NW * T
DUMMY = N_PAD - 8

_GDN = lax.GatherDimensionNumbers(
    offset_dims=(), collapsed_slice_dims=(0,), start_index_map=(0,))


def _splat(vec, lane):
  idx = jnp.full((16, 1), lane, jnp.int32)
  return lax.gather(vec, idx, _GDN, (1,),
                    mode=lax.GatherScatterMode.PROMISE_IN_BOUNDS)


_MESH = plsc.VectorSubcoreMesh(core_axis_name="c", subcore_axis_name="s")


@functools.partial(
    pl.kernel,
    mesh=_MESH,
    compiler_params=pltpu.CompilerParams(use_tc_tiling_on_sc=False),
    out_type=jax.ShapeDtypeStruct((2, N_PAD, DE), jnp.float32),
    scratch_types=[
        pltpu.VMEM((K,), jnp.int32),
        pltpu.VMEM((K,), jnp.int32),
        pltpu.VMEM((K,), jnp.float32),
        pltpu.VMEM((K, DE), jnp.float32),
        pltpu.VMEM((K, 16), jnp.float32),
        pltpu.VMEM((K, DE), jnp.float32),
        pltpu.VMEM((16,), jnp.float32),
        pltpu.VMEM_SHARED((N_PAD, DE), jnp.float32),
        pltpu.SemaphoreType.DMA,
        pltpu.SemaphoreType.DMA,
    ],
)
def _gat_sc(xh_hbm, adst_hbm, src_hbm, dst_hbm, ea_hbm, cvec_hbm, out_hbm,
            src_v, dst_v, ea_v, rows_v, adstr_v, obuf_v, cvec_v, accum,
            sem1, sem2):
  cid = lax.axis_index("c")
  sid = lax.axis_index("s")
  wid = sid * 2 + cid

  def zrow(e, _):
    for s in range(DE // 16):
      obuf_v[e, pl.ds(s * 16, 16)] = jnp.zeros((16,), jnp.float32)
    return 0
  lax.fori_loop(0, K, zrow, 0)

  def zcopy(j, _):
    pltpu.sync_copy(obuf_v, accum.at[pl.ds(sid * ROWS_SC + j * K, K)])
    return 0
  lax.fori_loop(0, ROWS_SC // K, zcopy, 0)
  rem = ROWS_SC % K
  if rem:
    pltpu.sync_copy(
        obuf_v.at[pl.ds(0, rem)],
        accum.at[pl.ds(sid * ROWS_SC + (ROWS_SC // K) * K, rem)])
  plsc.subcore_barrier()

  pltpu.sync_copy(cvec_hbm, cvec_v)
  cvec = cvec_v[...]
  lane = lax.iota(jnp.int32, 16)

  def chunk(j, _):
    base = j * K
    pltpu.sync_copy(src_hbm.at[wid, pl.ds(base, K)], src_v)
    pltpu.sync_copy(dst_hbm.at[wid, pl.ds(base, K)], dst_v)
    pltpu.sync_copy(ea_hbm.at[wid, pl.ds(base, K)], ea_v)
    g1 = pltpu.async_copy(xh_hbm.at[src_v], rows_v, sem1)
    g2 = pltpu.async_copy(adst_hbm.at[dst_v], adstr_v, sem2)
    g1.wait()
    g2.wait()

    def e16(q, _):
      ea16 = ea_v[pl.ds(q * 16, 16)]
      for l in range(16):
        e = q * 16 + l
        easp = _splat(ea16, l)
        asrc = rows_v[e, pl.ds(D, 16)]
        adst = adstr_v[e, :]
        alpha = asrc + adst + easp * cvec
        alpha = jnp.maximum(alpha, 0.2 * alpha)
        p = jnp.exp(alpha)
        p = jnp.where(lane < H, p, 0.0)
        obuf_v[e, pl.ds(D, 16)] = p
        for hd in range(H):
          psp = _splat(p, hd)
          obuf_v[e, pl.ds(hd * 16, 16)] = rows_v[e, pl.ds(hd * 16, 16)] * psp
      return 0
    lax.fori_loop(0, K // 16, e16, 0)

    pltpu.sync_copy(obuf_v, accum.at[dst_v], add=True)
    return 0
  lax.fori_loop(0, CH, chunk, 0)
  plsc.subcore_barrier()

  def wout(j, _):
    r = sid * ROWS_SC + j * K
    pltpu.sync_copy(accum.at[pl.ds(r, K)], out_hbm.at[cid, pl.ds(r, K)])
    return 0
  lax.fori_loop(0, ROWS_SC // K, wout, 0)
  if ROWS_SC % K:
    r = sid * ROWS_SC + (ROWS_SC // K) * K
    pltpu.sync_copy(accum.at[pl.ds(r, ROWS_SC % K)],
                    out_hbm.at[cid, pl.ds(r, ROWS_SC % K)])


def _dot(a, b):
  return jnp.dot(a, b, preferred_element_type=jnp.float32)


def _prep_body(x_ref, win_ref, wcat_ref, xh_ref, adst_ref):
  h = _dot(x_ref[...], win_ref[...])
  z = _dot(h, wcat_ref[...])
  xh_ref[...] = z[:, :DE]
  adst_ref[...] = z[:, DE:]


def _combine(acc_ref, erep_ref, bg_ref):
  a = acc_ref[0] + acc_ref[1]
  msg = a[:, :D]
  p = a[:, D:D + H]
  den = jnp.maximum(_dot(p, erep_ref[...]), 1e-30)
  return msg / den + bg_ref[...]


def _kan_tile(h, wc_ref, ws_ref):
  c1 = jnp.cos(h)
  s1 = jnp.sin(h)
  acc = _dot(c1, wc_ref[0]) + _dot(s1, ws_ref[0])
  cg, cgm1, sg, sgm1 = c1, jnp.ones_like(h), s1, jnp.zeros_like(h)
  for _ in range(1, G):
    cn = 2.0 * c1 * cg - cgm1
    sn = 2.0 * c1 * sg - sgm1
    cgm1, cg, sgm1, sg = cg, cn, sg, sn
    acc = acc + _dot(cg, wc_ref[_]) + _dot(sg, ws_ref[_])
  return acc


def _mid_body(acc_ref, bg_ref, wc_ref, ws_ref, wcat_ref, erep_ref,
              xh_ref, adst_ref):
  h = _combine(acc_ref, erep_ref, bg_ref)
  h2 = _kan_tile(h, wc_ref, ws_ref)
  z = _dot(h2, wcat_ref[...])
  xh_ref[...] = z[:, :DE]
  adst_ref[...] = z[:, DE:]


def _fin_body(bounds_ref, acc_ref, bg_ref, wc_ref, ws_ref, erep_ref,
              batch_ref, wout_ref, out_ref, pool_ref):
  i = pl.program_id(0)

  @pl.when(i == 0)
  def _():
    pool_ref[...] = jnp.full((NG, D), -jnp.inf, jnp.float32)

  h = _combine(acc_ref, erep_ref, bg_ref)
  h = _kan_tile(h, wc_ref, ws_ref)
  b = batch_ref[0]

  def body(g, _):
    mask = b == g
    part = jnp.max(jnp.where(mask, h, -jnp.inf), axis=0, keepdims=True)
    pool_ref[pl.ds(g, 1), :] = jnp.maximum(pool_ref[pl.ds(g, 1), :], part)
    return 0
  lax.fori_loop(bounds_ref[i, 0], bounds_ref[i, 1] + 1, body, 0)

  @pl.when(i == pl.num_programs(0) - 1)
  def _():
    pooled = pool_ref[...]
    pooled = jnp.where(jnp.isfinite(pooled), pooled, 0.0)
    out_ref[...] = _dot(pooled, wout_ref[...])


def _full(shape):
  zeros = tuple(0 for _ in shape)
  return pl.BlockSpec(shape, lambda i, z=zeros: z)


def kernel(x, edge_index, edge_weight, batch, W_in, W_gat, att_src, att_dst,
           W_edge, att_edge, b_gat, kan1, kan2, W_out):
  f32 = jnp.float32
  N = x.shape[0]
  E = edge_weight.shape[0]

  Wg3 = W_gat.reshape(D, H, C)
  A_src = jnp.einsum("khc,hc->kh", Wg3, att_src)
  A_dst = jnp.einsum("khc,hc->kh", Wg3, att_dst)
  pad8 = ((0, 0), (0, 8))
  Wcat = jnp.concatenate(
      [W_gat, jnp.pad(A_src, pad8), jnp.pad(A_dst, pad8)], axis=1)
  cvec16 = jnp.pad(jnp.sum(W_edge.reshape(H, C) * att_edge, axis=1), (0, 8))
  Erep = jnp.repeat(jnp.eye(H, dtype=f32), C, axis=1)
  Wc1 = jnp.transpose(kan1[0], (2, 1, 0))
  Ws1 = jnp.transpose(kan1[1], (2, 1, 0))
  Wc2 = jnp.transpose(kan2[0], (2, 1, 0))
  Ws2 = jnp.transpose(kan2[1], (2, 1, 0))
  bg = b_gat.reshape(1, D)

  loop = jnp.arange(N, dtype=jnp.int32)
  src = jnp.concatenate([edge_index[0].astype(jnp.int32), loop])
  dst = jnp.concatenate([edge_index[1].astype(jnp.int32), loop])
  ea = jnp.concatenate(
      [edge_weight, jnp.full((N,), jnp.mean(edge_weight), f32)])
  epad = E_PAD - (E + N)
  src = jnp.pad(src, (0, epad), constant_values=DUMMY).reshape(NW, T)
  dst = jnp.pad(dst, (0, epad), constant_values=DUMMY).reshape(NW, T)
  ea = jnp.pad(ea, (0, epad)).reshape(NW, T)

  x_pad = jnp.pad(x, ((0, N_PAD - N), (0, 0)))
  batch_i = jnp.pad(batch.astype(jnp.int32), (0, N_PAD - N),
                    constant_values=NG)
  batch_t = batch_i.reshape(N_TILES, ROWS, 1)
  bounds = jnp.stack(
      [jnp.min(batch_t[:, :, 0], axis=1),
       jnp.minimum(jnp.max(batch_t[:, :, 0], axis=1), NG - 1)],
      axis=1)

  row_spec = lambda d: pl.BlockSpec((ROWS, d), lambda i: (i, 0))
  acc_spec = pl.BlockSpec((2, ROWS, DE), lambda i: (0, i, 0))

  xh1, adst1 = pl.pallas_call(
      _prep_body,
      grid=(N_TILES,),
      in_specs=[row_spec(D), _full((D, D)), _full((D, DE + 16))],
      out_specs=[row_spec(DE), row_spec(16)],
      out_shape=[jax.ShapeDtypeStruct((N_PAD, DE), f32),
                 jax.ShapeDtypeStruct((N_PAD, 16), f32)],
  )(x_pad, W_in, Wcat)

  acc1 = _gat_sc(xh1, adst1, src, dst, ea, cvec16)

  xh2, adst2 = pl.pallas_call(
      _mid_body,
      grid=(N_TILES,),
      in_specs=[acc_spec, _full((1, D)), _full((G, D, D)), _full((G, D, D)),
                _full((D, DE + 16)), _full((H, D))],
      out_specs=[row_spec(DE), row_spec(16)],
      out_shape=[jax.ShapeDtypeStruct((N_PAD, DE), f32),
                 jax.ShapeDtypeStruct((N_PAD, 16), f32)],
  )(acc1, bg, Wc1, Ws1, Wcat, Erep)

  acc2 = _gat_sc(xh2, adst2, src, dst, ea, cvec16)

  out = pl.pallas_call(
      _fin_body,
      grid=(N_TILES,),
      in_specs=[pl.BlockSpec(memory_space=pltpu.SMEM), acc_spec,
                _full((1, D)), _full((G, D, D)), _full((G, D, D)),
                _full((H, D)), pl.BlockSpec((1, ROWS, 1), lambda i: (i, 0, 0)),
                _full((D, D))],
      out_specs=pl.BlockSpec((NG, D), lambda i: (0, 0)),
      out_shape=jax.ShapeDtypeStruct((NG, D), f32),
      scratch_shapes=[pltpu.VMEM((NG, D), f32)],
  )(bounds, acc2, bg, Wc2, Ws2, Erep, batch_t, W_out)

  return out
